# baseline (speedup 1.0000x reference)
import functools

import jax
import jax.numpy as jnp
import numpy as np
from jax import lax
from jax.experimental import pallas as pl
from jax.experimental.pallas import tpu as pltpu
from jax.experimental.pallas import tpu_sc as plsc

_INTERPRET = False

_MLP_INV = 1.0 / np.sqrt(1.0 + 1e-5)


def _fps_kernel(xyz_ref, inds_ref, nxyz_ref, *, npoint, P, R, L):
    x = xyz_ref[:, 0, :, :]
    y = xyz_ref[:, 1, :, :]
    z = xyz_ref[:, 2, :, :]
    B = x.shape[0]
    flat = (lax.broadcasted_iota(jnp.int32, (B, R, L), 1) * L
            + lax.broadcasted_iota(jnp.int32, (B, R, L), 2))

    def red(op, v):
        v = op(v, axis=2, keepdims=True)
        v = op(v, axis=1, keepdims=True)
        return v

    xall = xyz_ref[...]

    def coords_of(sel):
        t = jnp.where(sel[:, None, :, :], xall, 0.0)
        s = jnp.sum(jnp.sum(t, axis=3, keepdims=True), axis=2, keepdims=True)
        return s[:, 0], s[:, 1], s[:, 2]

    def store_centroid(i, lx, ly, lz):
        nxyz_ref[:, pl.ds(i, 1), :] = jnp.concatenate([lx, ly, lz], axis=2)

    inds_ref[:, 0:1, :] = jnp.zeros((B, 1, 1), jnp.int32)

    def body(i, carry):
        dists, last = carry
        sel = flat == last
        lx, ly, lz = coords_of(sel)
        store_centroid(i - 1, lx, ly, lz)
        dx = x - lx
        dy = y - ly
        dz = z - lz
        d = (dx * dx + dy * dy) + dz * dz
        dists = jnp.minimum(dists, d)
        m = red(jnp.max, dists)
        nxt = red(jnp.min, jnp.where(dists == m, flat, P))
        inds_ref[:, pl.ds(i, 1), :] = nxt[:, :, 0:1]
        return dists, nxt

    dists0 = jnp.full((B, R, L), 1e10, jnp.float32)
    last0 = jnp.zeros((B, 1, 1), jnp.int32)
    _, last = lax.fori_loop(1, npoint, body, (dists0, last0))
    sel = flat == last
    lx, ly, lz = coords_of(sel)
    store_centroid(npoint - 1, lx, ly, lz)


def _fps(xyz, npoint):
    B, P, _ = xyz.shape
    R = 8
    L = P // R
    xyz_r = jnp.transpose(xyz, (0, 2, 1)).reshape(B, 3, R, L)
    inds, nxyz = pl.pallas_call(
        functools.partial(_fps_kernel, npoint=npoint, P=P, R=R, L=L),
        out_shape=[
            jax.ShapeDtypeStruct((B, npoint, 1), jnp.int32),
            jax.ShapeDtypeStruct((B, npoint, 3), jnp.float32),
        ],
        interpret=_INTERPRET,
    )(xyz_r)
    return inds, nxyz


def _ball_gather_sc(xs, ys, zs, cx, cy, cz, radius, nsample):
    B, P = xs.shape
    ST = cx.shape[0]
    SPB = ST // B
    NC, NS = 2, 16
    NW = NC * NS
    CPW = ST // NW
    UNROLL = 8
    PV = P // (16 * UNROLL)
    PAD = nsample + 16 * UNROLL
    OUTW = CPW * nsample * 3
    r2 = radius * radius

    mesh = plsc.VectorSubcoreMesh(
        core_axis_name="c", subcore_axis_name="s",
        num_cores=NC, num_subcores=NS)

    @functools.partial(
        pl.kernel,
        out_type=jax.ShapeDtypeStruct((ST * nsample * 3,), jnp.float32),
        mesh=mesh,
        compiler_params=pltpu.CompilerParams(needs_layout_passes=False),
        scratch_types=[
            pltpu.VMEM((P,), jnp.float32),
            pltpu.VMEM((P,), jnp.float32),
            pltpu.VMEM((P,), jnp.float32),
            pltpu.VMEM((CPW,), jnp.float32),
            pltpu.VMEM((CPW,), jnp.float32),
            pltpu.VMEM((CPW,), jnp.float32),
            pltpu.VMEM((PAD,), jnp.int32),
            pltpu.VMEM((OUTW,), jnp.float32),
        ],
    )
    def k(xs_h, ys_h, zs_h, cx_h, cy_h, cz_h, out_h,
          xs_v, ys_v, zs_v, cx_v, cy_v, cz_v, idx_v, out_v):
        wid = lax.axis_index("s") * NC + lax.axis_index("c")
        base = wid * CPW
        b = base // SPB
        pltpu.sync_copy(xs_h.at[b], xs_v)
        pltpu.sync_copy(ys_h.at[b], ys_v)
        pltpu.sync_copy(zs_h.at[b], zs_v)
        pltpu.sync_copy(cx_h.at[pl.ds(base, CPW)], cx_v)
        pltpu.sync_copy(cy_h.at[pl.ds(base, CPW)], cy_v)
        pltpu.sync_copy(cz_h.at[pl.ds(base, CPW)], cz_v)

        lane = lax.broadcasted_iota(jnp.int32, (16,), 0)
        zero16 = jnp.zeros((16,), jnp.int32)

        def per_centroid(j, carry):
            jv = zero16 + j
            cxv = plsc.load_gather(cx_v, [jv])
            cyv = plsc.load_gather(cy_v, [jv])
            czv = plsc.load_gather(cz_v, [jv])

            def cond(st):
                i, cnt = st
                return jnp.logical_and(i < PV, cnt < nsample)

            def sbody(st):
                i, cnt = st
                base_off = i * (16 * UNROLL)
                masks = []
                for u in range(UNROLL):
                    off = base_off + u * 16
                    xv = xs_v[pl.ds(off, 16)]
                    yv = ys_v[pl.ds(off, 16)]
                    zv = zs_v[pl.ds(off, 16)]
                    dx = xv - cxv
                    dy = yv - cyv
                    dz = zv - czv
                    d2 = (dx * dx + dy * dy) + dz * dz
                    masks.append(d2 < r2)
                for u in range(UNROLL):
                    off = base_off + u * 16
                    mk = masks[u]
                    plsc.store_compressed(
                        idx_v.at[pl.ds(cnt, 16)], lane + off, mask=mk)
                    cnt = cnt + jnp.sum(mk.astype(jnp.int32))
                return i + 1, cnt

            _, cnt = lax.while_loop(cond, sbody, (jnp.int32(0), jnp.int32(0)))

            cntv = zero16 + cnt
            head = idx_v[pl.ds(0, 16)]
            first = jnp.min(jnp.where(lane < cntv, head, jnp.int32(2 ** 30)))
            firstv = zero16 + jnp.where(cnt > 0, first, jnp.int32(0))
            for q in range(nsample // 16):
                pos = lane + q * 16
                plsc.store_scatter(idx_v, [pos], firstv, mask=pos >= cntv)
            for g in range(nsample // 16):
                pos = lane + g * 16
                sel = idx_v[pl.ds(g * 16, 16)]
                gx = plsc.load_gather(xs_v, [sel])
                gy = plsc.load_gather(ys_v, [sel])
                gz = plsc.load_gather(zs_v, [sel])
                fl = (jv * nsample + pos) * 3
                plsc.store_scatter(out_v, [fl], gx)
                plsc.store_scatter(out_v, [fl + 1], gy)
                plsc.store_scatter(out_v, [fl + 2], gz)
            return carry

        lax.fori_loop(0, CPW, per_centroid, 0)
        pltpu.sync_copy(out_v, out_h.at[pl.ds(base * nsample * 3, OUTW)])

    flat = k(xs, ys, zs, cx, cy, cz)
    return flat.reshape(ST, nsample, 3)


def _sa_grouped_kernel(g_ref, c_ref, *wrefs, out_ref, r, nsample, S_b):
    nlayers = len(wrefs) // 3
    c = c_ref[0]
    rows = S_b * nsample
    g2 = g_ref[...].reshape(rows, 3)
    crep = jnp.broadcast_to(c[:, None, :], (S_b, nsample, 3)).reshape(rows, 3)
    h = (g2 - crep) / r
    for li in range(nlayers):
        w = wrefs[3 * li][...]
        g = wrefs[3 * li + 1][...]
        b = wrefs[3 * li + 2][...]
        h = jnp.dot(h.astype(jnp.bfloat16), w.astype(jnp.bfloat16),
                    preferred_element_type=jnp.float32)
        h = h * _MLP_INV
        h = g * h + b
        h = jnp.maximum(h, 0.0)
    cout = h.shape[1]
    out_ref[0] = jnp.max(h.reshape(S_b, nsample, cout), axis=1)


def _sa_from_grouped(grouped, new_xyz, radius, nsample, layers, S_b):
    ST = grouped.shape[0]
    B, S, _ = new_xyz.shape
    nblk = S // S_b

    wargs = []
    wspecs = []
    for (w, gamma, beta) in layers:
        wargs += [jnp.transpose(w), gamma[None, :], beta[None, :]]
        wspecs += [
            pl.BlockSpec((w.shape[1], w.shape[0]), lambda b, s: (0, 0)),
            pl.BlockSpec((1, gamma.shape[0]), lambda b, s: (0, 0)),
            pl.BlockSpec((1, beta.shape[0]), lambda b, s: (0, 0)),
        ]
    cout = layers[-1][0].shape[0]

    body = functools.partial(
        _sa_grouped_kernel, r=radius, nsample=nsample, S_b=S_b)

    def wrapped(g_r, c_r, *wr):
        body(g_r, c_r, *wr[:-1], out_ref=wr[-1])

    return pl.pallas_call(
        wrapped,
        grid=(B, nblk),
        in_specs=[
            pl.BlockSpec((S_b, nsample, 3), lambda b, s: (b * nblk + s, 0, 0)),
            pl.BlockSpec((1, S_b, 3), lambda b, s: (b, s, 0)),
        ] + wspecs,
        out_specs=pl.BlockSpec((1, S_b, cout), lambda b, s: (b, s, 0)),
        out_shape=jax.ShapeDtypeStruct((B, S, cout), jnp.float32),
        interpret=_INTERPRET,
    )(grouped, new_xyz, *wargs)


def _cumsum_lanes(x):
    rows, L = x.shape
    s = 1
    while s < L:
        x = x + jnp.concatenate(
            [jnp.zeros((rows, s), x.dtype), x[:, :L - s]], axis=1)
        s *= 2
    return x


def _sa_kernel(xyzt_ref, x_ref, c_ref, *wrefs, out_ref, r, r2, nsample, Pc, S_b):
    P = xyzt_ref.shape[2]
    cin = x_ref.shape[2]
    rows = S_b * nsample
    nlayers = len(wrefs) // 3

    c = c_ref[0]
    cx = c[:, 0:1]
    cy = c[:, 1:2]
    cz = c[:, 2:3]

    g_acc = jnp.zeros((rows, cin), jnp.float32)
    base = jnp.zeros((S_b, 1), jnp.int32)
    for j in range(P // Pc):
        px = xyzt_ref[0, 0:1, j * Pc:(j + 1) * Pc]
        py = xyzt_ref[0, 1:2, j * Pc:(j + 1) * Pc]
        pz = xyzt_ref[0, 2:3, j * Pc:(j + 1) * Pc]
        dx = px - cx
        dy = py - cy
        dz = pz - cz
        d2 = (dx * dx + dy * dy) + dz * dz
        m = d2 < r2
        rank = _cumsum_lanes(m.astype(jnp.int32)) + base
        base = rank[:, Pc - 1:Pc]
        ok = m & (rank <= nsample)
        kk = lax.broadcasted_iota(jnp.int32, (S_b, nsample, Pc), 1) + 1
        oh = (rank[:, None, :] == kk) & ok[:, None, :]
        ohf = oh.reshape(rows, Pc).astype(jnp.float32)
        xc = x_ref[0, j * Pc:(j + 1) * Pc, :]
        g_acc = g_acc + jnp.dot(ohf, xc, preferred_element_type=jnp.float32,
                                precision=lax.Precision.HIGHEST)

    total = base
    crep = jnp.broadcast_to(c[:, None, :], (S_b, nsample, 3)).reshape(rows, 3)
    axyz = (g_acc[:, 0:3] - crep) / r
    h = jnp.concatenate([axyz, g_acc[:, 3:]], axis=1) if cin > 3 else axyz
    for li in range(nlayers):
        w = wrefs[3 * li][...]
        g = wrefs[3 * li + 1][...]
        b = wrefs[3 * li + 2][...]
        h = jnp.dot(h.astype(jnp.bfloat16), w.astype(jnp.bfloat16),
                    preferred_element_type=jnp.float32)
        h = h * _MLP_INV
        h = g * h + b
        h = jnp.maximum(h, 0.0)

    cout = h.shape[1]
    kidx = lax.broadcasted_iota(jnp.int32, (rows, 1), 0) % nsample
    tot_e = jnp.broadcast_to(total[:, None, :], (S_b, nsample, 1)).reshape(rows, 1)
    valid = kidx < jnp.minimum(tot_e, nsample)
    hm = jnp.where(valid, h, -jnp.inf)
    out_ref[0] = jnp.max(hm.reshape(S_b, nsample, cout), axis=1)


def _sa(xyz, feats, new_xyz, radius, nsample, layers, S_b, Pc):
    B, P, _ = xyz.shape
    S = new_xyz.shape[1]
    xyzt = jnp.transpose(xyz, (0, 2, 1))
    x_in = xyz if feats is None else jnp.concatenate([xyz, feats], axis=2)
    cin = x_in.shape[2]
    r = np.float32(radius)

    wargs = []
    wspecs = []
    for li, (w, gamma, beta) in enumerate(layers):
        wt = jnp.transpose(w)
        wargs += [wt, gamma[None, :], beta[None, :]]
        wspecs += [
            pl.BlockSpec(wt.shape, lambda b, s: (0, 0)),
            pl.BlockSpec((1, gamma.shape[0]), lambda b, s: (0, 0)),
            pl.BlockSpec((1, beta.shape[0]), lambda b, s: (0, 0)),
        ]
    cout = layers[-1][0].shape[0]

    body = functools.partial(
        _sa_kernel, r=radius, r2=radius * radius, nsample=nsample,
        Pc=Pc, S_b=S_b)

    def wrapped(xyzt_r, x_r, c_r, *wr):
        body(xyzt_r, x_r, c_r, *wr[:-1], out_ref=wr[-1])

    return pl.pallas_call(
        wrapped,
        grid=(B, S // S_b),
        in_specs=[
            pl.BlockSpec((1, 3, P), lambda b, s: (b, 0, 0)),
            pl.BlockSpec((1, P, cin), lambda b, s: (b, 0, 0)),
            pl.BlockSpec((1, S_b, 3), lambda b, s: (b, s, 0)),
        ] + wspecs,
        out_specs=pl.BlockSpec((1, S_b, cout), lambda b, s: (b, s, 0)),
        out_shape=jax.ShapeDtypeStruct((B, S, cout), jnp.float32),
        interpret=_INTERPRET,
    )(xyzt, x_in, new_xyz, *wargs)


def _fp_kernel(q_ref, kt_ref, f1_ref, f2_ref, *wrefs, out_ref):
    nlayers = len(wrefs) // 3
    q = q_ref[0]
    qx = q[:, 0:1]
    qy = q[:, 1:2]
    qz = q[:, 2:3]
    kx = kt_ref[0, 0:1, :]
    ky = kt_ref[0, 1:2, :]
    kz = kt_ref[0, 2:3, :]
    dx = qx - kx
    dy = qy - ky
    dz = qz - kz
    d2 = (dx * dx + dy * dy) + dz * dz
    S, K = d2.shape
    iota = lax.broadcasted_iota(jnp.int32, (S, K), 1)

    d = d2
    wmat = jnp.zeros((S, K), jnp.float32)
    wsum = jnp.zeros((S, 1), jnp.float32)
    for _ in range(3):
        mval = jnp.min(d, axis=1, keepdims=True)
        idx = jnp.min(jnp.where(d == mval, iota, K), axis=1, keepdims=True)
        onehot = iota == idx
        wj = 1.0 / (mval + 1e-8)
        wmat = wmat + jnp.where(onehot, wj, 0.0)
        wsum = wsum + wj
        d = jnp.where(onehot, 1e30, d)
    wmat = wmat / wsum

    interp = jnp.dot(wmat, f2_ref[0], preferred_element_type=jnp.float32,
                     precision=lax.Precision.HIGHEST)
    h = jnp.concatenate([interp, f1_ref[0]], axis=1)
    for li in range(nlayers):
        w = wrefs[3 * li][...]
        g = wrefs[3 * li + 1][...]
        b = wrefs[3 * li + 2][...]
        h = jnp.dot(h.astype(jnp.bfloat16), w.astype(jnp.bfloat16),
                    preferred_element_type=jnp.float32)
        h = h * _MLP_INV
        h = g * h + b
        h = jnp.maximum(h, 0.0)
    out_ref[0] = h


def _fp(unknown, known, feats1, feats2, layers):
    B, S, _ = unknown.shape
    K = known.shape[1]
    c1 = feats1.shape[2]
    c2 = feats2.shape[2]
    knownt = jnp.transpose(known, (0, 2, 1))

    wargs = []
    wspecs = []
    for (w, gamma, beta) in layers:
        wargs += [jnp.transpose(w), gamma[None, :], beta[None, :]]
        wspecs += [
            pl.BlockSpec((w.shape[1], w.shape[0]), lambda b: (0, 0)),
            pl.BlockSpec((1, gamma.shape[0]), lambda b: (0, 0)),
            pl.BlockSpec((1, beta.shape[0]), lambda b: (0, 0)),
        ]
    cout = layers[-1][0].shape[0]

    def wrapped(q_r, kt_r, f1_r, f2_r, *wr):
        _fp_kernel(q_r, kt_r, f1_r, f2_r, *wr[:-1], out_ref=wr[-1])

    return pl.pallas_call(
        wrapped,
        grid=(B,),
        in_specs=[
            pl.BlockSpec((1, S, 3), lambda b: (b, 0, 0)),
            pl.BlockSpec((1, 3, K), lambda b: (b, 0, 0)),
            pl.BlockSpec((1, S, c1), lambda b: (b, 0, 0)),
            pl.BlockSpec((1, K, c2), lambda b: (b, 0, 0)),
        ] + wspecs,
        out_specs=pl.BlockSpec((1, S, cout), lambda b: (b, 0, 0)),
        out_shape=jax.ShapeDtypeStruct((B, S, cout), jnp.float32),
        interpret=_INTERPRET,
    )(unknown, knownt, feats1, feats2, *wargs)


def kernel(pc, params):
    xyz = pc[..., :3]
    inds1, c1 = _fps(xyz, 1024)
    grouped1 = _ball_gather_sc(
        xyz[:, :, 0], xyz[:, :, 1], xyz[:, :, 2],
        c1[:, :, 0].reshape(-1), c1[:, :, 1].reshape(-1),
        c1[:, :, 2].reshape(-1), 0.1, 64)
    f1 = _sa_from_grouped(grouped1, c1, 0.1, 64, params['sa1'], S_b=32)
    _, c2 = _fps(c1, 512)
    f2 = _sa(c1, f1, c2, 0.2, 32, params['sa2'], S_b=16, Pc=1024)
    _, c3 = _fps(c2, 256)
    f3 = _sa(c2, f2, c3, 0.4, 16, params['sa3'], S_b=32, Pc=512)
    latf = _fp(c2, c3, f2, f3, params['fp2'])
    return c2, latf, inds1[:, :512, 0]


# speedup vs baseline: 1.0013x; 1.0013x over previous
import functools

import jax
import jax.numpy as jnp
import numpy as np
from jax import lax
from jax.experimental import pallas as pl
from jax.experimental.pallas import tpu as pltpu
from jax.experimental.pallas import tpu_sc as plsc

_INTERPRET = False

_MLP_INV = 1.0 / np.sqrt(1.0 + 1e-5)


def _fps_kernel(xyz_ref, inds_ref, nxyz_ref, *, npoint, P, R, L):
    x = xyz_ref[:, 0, :, :]
    y = xyz_ref[:, 1, :, :]
    z = xyz_ref[:, 2, :, :]
    B = x.shape[0]
    flat = (lax.broadcasted_iota(jnp.int32, (B, R, L), 1) * L
            + lax.broadcasted_iota(jnp.int32, (B, R, L), 2))

    def red(op, v):
        v = op(v, axis=2, keepdims=True)
        v = op(v, axis=1, keepdims=True)
        return v

    xall = xyz_ref[...]

    def coords_of(sel):
        t = jnp.where(sel[:, None, :, :], xall, 0.0)
        s = jnp.sum(jnp.sum(t, axis=3, keepdims=True), axis=2, keepdims=True)
        return s[:, 0], s[:, 1], s[:, 2]

    def store_centroid(i, lx, ly, lz):
        nxyz_ref[:, pl.ds(i, 1), :] = jnp.concatenate([lx, ly, lz], axis=2)

    inds_ref[:, 0:1, :] = jnp.zeros((B, 1, 1), jnp.int32)

    def body(i, carry):
        dists, last = carry
        sel = flat == last
        lx, ly, lz = coords_of(sel)
        store_centroid(i - 1, lx, ly, lz)
        dx = x - lx
        dy = y - ly
        dz = z - lz
        d = (dx * dx + dy * dy) + dz * dz
        dists = jnp.minimum(dists, d)
        m = red(jnp.max, dists)
        nxt = red(jnp.min, jnp.where(dists == m, flat, P))
        inds_ref[:, pl.ds(i, 1), :] = nxt[:, :, 0:1]
        return dists, nxt

    dists0 = jnp.full((B, R, L), 1e10, jnp.float32)
    last0 = jnp.zeros((B, 1, 1), jnp.int32)
    _, last = lax.fori_loop(1, npoint, body, (dists0, last0))
    sel = flat == last
    lx, ly, lz = coords_of(sel)
    store_centroid(npoint - 1, lx, ly, lz)


def _fps(xyz, npoint):
    B, P, _ = xyz.shape
    R = 8
    L = P // R
    xyz_r = jnp.transpose(xyz, (0, 2, 1)).reshape(B, 3, R, L)
    inds, nxyz = pl.pallas_call(
        functools.partial(_fps_kernel, npoint=npoint, P=P, R=R, L=L),
        out_shape=[
            jax.ShapeDtypeStruct((B, npoint, 1), jnp.int32),
            jax.ShapeDtypeStruct((B, npoint, 3), jnp.float32),
        ],
        interpret=_INTERPRET,
    )(xyz_r)
    return inds, nxyz


def _ball_gather_sc(xs, ys, zs, cx, cy, cz, radius, nsample):
    B, P = xs.shape
    ST = cx.shape[0]
    SPB = ST // B
    NC, NS = 2, 16
    NW = NC * NS
    CPW = ST // NW
    UNROLL = 8
    PV = P // (16 * UNROLL)
    PAD = nsample + 16 * UNROLL
    OUTW = CPW * nsample * 3
    r2 = radius * radius

    mesh = plsc.VectorSubcoreMesh(
        core_axis_name="c", subcore_axis_name="s",
        num_cores=NC, num_subcores=NS)

    @functools.partial(
        pl.kernel,
        out_type=jax.ShapeDtypeStruct((ST * nsample * 3,), jnp.float32),
        mesh=mesh,
        compiler_params=pltpu.CompilerParams(needs_layout_passes=False),
        scratch_types=[
            pltpu.VMEM((P,), jnp.float32),
            pltpu.VMEM((P,), jnp.float32),
            pltpu.VMEM((P,), jnp.float32),
            pltpu.VMEM((CPW,), jnp.float32),
            pltpu.VMEM((CPW,), jnp.float32),
            pltpu.VMEM((CPW,), jnp.float32),
            pltpu.VMEM((PAD,), jnp.int32),
            pltpu.VMEM((OUTW,), jnp.float32),
        ],
    )
    def k(xs_h, ys_h, zs_h, cx_h, cy_h, cz_h, out_h,
          xs_v, ys_v, zs_v, cx_v, cy_v, cz_v, idx_v, out_v):
        wid = lax.axis_index("s") * NC + lax.axis_index("c")
        base = wid * CPW
        b = base // SPB
        pltpu.sync_copy(xs_h.at[b], xs_v)
        pltpu.sync_copy(ys_h.at[b], ys_v)
        pltpu.sync_copy(zs_h.at[b], zs_v)
        pltpu.sync_copy(cx_h.at[pl.ds(base, CPW)], cx_v)
        pltpu.sync_copy(cy_h.at[pl.ds(base, CPW)], cy_v)
        pltpu.sync_copy(cz_h.at[pl.ds(base, CPW)], cz_v)

        lane = lax.broadcasted_iota(jnp.int32, (16,), 0)
        zero16 = jnp.zeros((16,), jnp.int32)

        def per_centroid(j, carry):
            jv = zero16 + j
            cxv = plsc.load_gather(cx_v, [jv])
            cyv = plsc.load_gather(cy_v, [jv])
            czv = plsc.load_gather(cz_v, [jv])

            def cond(st):
                i, cnt = st
                return jnp.logical_and(i < PV, cnt < nsample)

            def sbody(st):
                i, cnt = st
                base_off = i * (16 * UNROLL)
                masks = []
                for u in range(UNROLL):
                    off = base_off + u * 16
                    xv = xs_v[pl.ds(off, 16)]
                    yv = ys_v[pl.ds(off, 16)]
                    zv = zs_v[pl.ds(off, 16)]
                    dx = xv - cxv
                    dy = yv - cyv
                    dz = zv - czv
                    d2 = (dx * dx + dy * dy) + dz * dz
                    masks.append(d2 < r2)
                pcs = [jnp.sum(mk.astype(jnp.int32)) for mk in masks]
                offs = []
                for u in range(UNROLL):
                    offs.append(cnt)
                    cnt = cnt + pcs[u]
                for u in range(UNROLL):
                    off = base_off + u * 16
                    plsc.store_compressed(
                        idx_v.at[pl.ds(offs[u], 16)], lane + off,
                        mask=masks[u])
                return i + 1, cnt

            _, cnt = lax.while_loop(cond, sbody, (jnp.int32(0), jnp.int32(0)))

            cntv = zero16 + cnt
            head = idx_v[pl.ds(0, 16)]
            first = jnp.min(jnp.where(lane < cntv, head, jnp.int32(2 ** 30)))
            firstv = zero16 + jnp.where(cnt > 0, first, jnp.int32(0))
            for q in range(nsample // 16):
                pos = lane + q * 16
                plsc.store_scatter(idx_v, [pos], firstv, mask=pos >= cntv)
            for g in range(nsample // 16):
                pos = lane + g * 16
                sel = idx_v[pl.ds(g * 16, 16)]
                gx = plsc.load_gather(xs_v, [sel])
                gy = plsc.load_gather(ys_v, [sel])
                gz = plsc.load_gather(zs_v, [sel])
                fl = (jv * nsample + pos) * 3
                plsc.store_scatter(out_v, [fl], gx)
                plsc.store_scatter(out_v, [fl + 1], gy)
                plsc.store_scatter(out_v, [fl + 2], gz)
            return carry

        lax.fori_loop(0, CPW, per_centroid, 0)
        pltpu.sync_copy(out_v, out_h.at[pl.ds(base * nsample * 3, OUTW)])

    flat = k(xs, ys, zs, cx, cy, cz)
    return flat.reshape(ST, nsample, 3)


def _sa_grouped_kernel(g_ref, c_ref, *wrefs, out_ref, r, nsample, S_b):
    nlayers = len(wrefs) // 3
    c = c_ref[0]
    rows = S_b * nsample
    g2 = g_ref[...].reshape(rows, 3)
    crep = jnp.broadcast_to(c[:, None, :], (S_b, nsample, 3)).reshape(rows, 3)
    h = (g2 - crep) / r
    for li in range(nlayers):
        w = wrefs[3 * li][...]
        g = wrefs[3 * li + 1][...]
        b = wrefs[3 * li + 2][...]
        h = jnp.dot(h.astype(jnp.bfloat16), w.astype(jnp.bfloat16),
                    preferred_element_type=jnp.float32)
        h = h * _MLP_INV
        h = g * h + b
        h = jnp.maximum(h, 0.0)
    cout = h.shape[1]
    out_ref[0] = jnp.max(h.reshape(S_b, nsample, cout), axis=1)


def _sa_from_grouped(grouped, new_xyz, radius, nsample, layers, S_b):
    ST = grouped.shape[0]
    B, S, _ = new_xyz.shape
    nblk = S // S_b

    wargs = []
    wspecs = []
    for (w, gamma, beta) in layers:
        wargs += [jnp.transpose(w), gamma[None, :], beta[None, :]]
        wspecs += [
            pl.BlockSpec((w.shape[1], w.shape[0]), lambda b, s: (0, 0)),
            pl.BlockSpec((1, gamma.shape[0]), lambda b, s: (0, 0)),
            pl.BlockSpec((1, beta.shape[0]), lambda b, s: (0, 0)),
        ]
    cout = layers[-1][0].shape[0]

    body = functools.partial(
        _sa_grouped_kernel, r=radius, nsample=nsample, S_b=S_b)

    def wrapped(g_r, c_r, *wr):
        body(g_r, c_r, *wr[:-1], out_ref=wr[-1])

    return pl.pallas_call(
        wrapped,
        grid=(B, nblk),
        in_specs=[
            pl.BlockSpec((S_b, nsample, 3), lambda b, s: (b * nblk + s, 0, 0)),
            pl.BlockSpec((1, S_b, 3), lambda b, s: (b, s, 0)),
        ] + wspecs,
        out_specs=pl.BlockSpec((1, S_b, cout), lambda b, s: (b, s, 0)),
        out_shape=jax.ShapeDtypeStruct((B, S, cout), jnp.float32),
        interpret=_INTERPRET,
    )(grouped, new_xyz, *wargs)


def _cumsum_lanes(x):
    rows, L = x.shape
    s = 1
    while s < L:
        x = x + jnp.concatenate(
            [jnp.zeros((rows, s), x.dtype), x[:, :L - s]], axis=1)
        s *= 2
    return x


def _sa_kernel(xyzt_ref, x_ref, c_ref, *wrefs, out_ref, r, r2, nsample, Pc, S_b):
    P = xyzt_ref.shape[2]
    cin = x_ref.shape[2]
    rows = S_b * nsample
    nlayers = len(wrefs) // 3

    c = c_ref[0]
    cx = c[:, 0:1]
    cy = c[:, 1:2]
    cz = c[:, 2:3]

    g_acc = jnp.zeros((rows, cin), jnp.float32)
    base = jnp.zeros((S_b, 1), jnp.int32)
    for j in range(P // Pc):
        px = xyzt_ref[0, 0:1, j * Pc:(j + 1) * Pc]
        py = xyzt_ref[0, 1:2, j * Pc:(j + 1) * Pc]
        pz = xyzt_ref[0, 2:3, j * Pc:(j + 1) * Pc]
        dx = px - cx
        dy = py - cy
        dz = pz - cz
        d2 = (dx * dx + dy * dy) + dz * dz
        m = d2 < r2
        rank = _cumsum_lanes(m.astype(jnp.int32)) + base
        base = rank[:, Pc - 1:Pc]
        ok = m & (rank <= nsample)
        kk = lax.broadcasted_iota(jnp.int32, (S_b, nsample, Pc), 1) + 1
        oh = (rank[:, None, :] == kk) & ok[:, None, :]
        ohf = oh.reshape(rows, Pc).astype(jnp.float32)
        xc = x_ref[0, j * Pc:(j + 1) * Pc, :]
        g_acc = g_acc + jnp.dot(ohf, xc, preferred_element_type=jnp.float32,
                                precision=lax.Precision.HIGHEST)

    total = base
    crep = jnp.broadcast_to(c[:, None, :], (S_b, nsample, 3)).reshape(rows, 3)
    axyz = (g_acc[:, 0:3] - crep) / r
    h = jnp.concatenate([axyz, g_acc[:, 3:]], axis=1) if cin > 3 else axyz
    for li in range(nlayers):
        w = wrefs[3 * li][...]
        g = wrefs[3 * li + 1][...]
        b = wrefs[3 * li + 2][...]
        h = jnp.dot(h.astype(jnp.bfloat16), w.astype(jnp.bfloat16),
                    preferred_element_type=jnp.float32)
        h = h * _MLP_INV
        h = g * h + b
        h = jnp.maximum(h, 0.0)

    cout = h.shape[1]
    kidx = lax.broadcasted_iota(jnp.int32, (rows, 1), 0) % nsample
    tot_e = jnp.broadcast_to(total[:, None, :], (S_b, nsample, 1)).reshape(rows, 1)
    valid = kidx < jnp.minimum(tot_e, nsample)
    hm = jnp.where(valid, h, -jnp.inf)
    out_ref[0] = jnp.max(hm.reshape(S_b, nsample, cout), axis=1)


def _sa(xyz, feats, new_xyz, radius, nsample, layers, S_b, Pc):
    B, P, _ = xyz.shape
    S = new_xyz.shape[1]
    xyzt = jnp.transpose(xyz, (0, 2, 1))
    x_in = xyz if feats is None else jnp.concatenate([xyz, feats], axis=2)
    cin = x_in.shape[2]
    r = np.float32(radius)

    wargs = []
    wspecs = []
    for li, (w, gamma, beta) in enumerate(layers):
        wt = jnp.transpose(w)
        wargs += [wt, gamma[None, :], beta[None, :]]
        wspecs += [
            pl.BlockSpec(wt.shape, lambda b, s: (0, 0)),
            pl.BlockSpec((1, gamma.shape[0]), lambda b, s: (0, 0)),
            pl.BlockSpec((1, beta.shape[0]), lambda b, s: (0, 0)),
        ]
    cout = layers[-1][0].shape[0]

    body = functools.partial(
        _sa_kernel, r=radius, r2=radius * radius, nsample=nsample,
        Pc=Pc, S_b=S_b)

    def wrapped(xyzt_r, x_r, c_r, *wr):
        body(xyzt_r, x_r, c_r, *wr[:-1], out_ref=wr[-1])

    return pl.pallas_call(
        wrapped,
        grid=(B, S // S_b),
        in_specs=[
            pl.BlockSpec((1, 3, P), lambda b, s: (b, 0, 0)),
            pl.BlockSpec((1, P, cin), lambda b, s: (b, 0, 0)),
            pl.BlockSpec((1, S_b, 3), lambda b, s: (b, s, 0)),
        ] + wspecs,
        out_specs=pl.BlockSpec((1, S_b, cout), lambda b, s: (b, s, 0)),
        out_shape=jax.ShapeDtypeStruct((B, S, cout), jnp.float32),
        interpret=_INTERPRET,
    )(xyzt, x_in, new_xyz, *wargs)


def _fp_kernel(q_ref, kt_ref, f1_ref, f2_ref, *wrefs, out_ref):
    nlayers = len(wrefs) // 3
    q = q_ref[0]
    qx = q[:, 0:1]
    qy = q[:, 1:2]
    qz = q[:, 2:3]
    kx = kt_ref[0, 0:1, :]
    ky = kt_ref[0, 1:2, :]
    kz = kt_ref[0, 2:3, :]
    dx = qx - kx
    dy = qy - ky
    dz = qz - kz
    d2 = (dx * dx + dy * dy) + dz * dz
    S, K = d2.shape
    iota = lax.broadcasted_iota(jnp.int32, (S, K), 1)

    d = d2
    wmat = jnp.zeros((S, K), jnp.float32)
    wsum = jnp.zeros((S, 1), jnp.float32)
    for _ in range(3):
        mval = jnp.min(d, axis=1, keepdims=True)
        idx = jnp.min(jnp.where(d == mval, iota, K), axis=1, keepdims=True)
        onehot = iota == idx
        wj = 1.0 / (mval + 1e-8)
        wmat = wmat + jnp.where(onehot, wj, 0.0)
        wsum = wsum + wj
        d = jnp.where(onehot, 1e30, d)
    wmat = wmat / wsum

    interp = jnp.dot(wmat, f2_ref[0], preferred_element_type=jnp.float32,
                     precision=lax.Precision.HIGHEST)
    h = jnp.concatenate([interp, f1_ref[0]], axis=1)
    for li in range(nlayers):
        w = wrefs[3 * li][...]
        g = wrefs[3 * li + 1][...]
        b = wrefs[3 * li + 2][...]
        h = jnp.dot(h.astype(jnp.bfloat16), w.astype(jnp.bfloat16),
                    preferred_element_type=jnp.float32)
        h = h * _MLP_INV
        h = g * h + b
        h = jnp.maximum(h, 0.0)
    out_ref[0] = h


def _fp(unknown, known, feats1, feats2, layers):
    B, S, _ = unknown.shape
    K = known.shape[1]
    c1 = feats1.shape[2]
    c2 = feats2.shape[2]
    knownt = jnp.transpose(known, (0, 2, 1))

    wargs = []
    wspecs = []
    for (w, gamma, beta) in layers:
        wargs += [jnp.transpose(w), gamma[None, :], beta[None, :]]
        wspecs += [
            pl.BlockSpec((w.shape[1], w.shape[0]), lambda b: (0, 0)),
            pl.BlockSpec((1, gamma.shape[0]), lambda b: (0, 0)),
            pl.BlockSpec((1, beta.shape[0]), lambda b: (0, 0)),
        ]
    cout = layers[-1][0].shape[0]

    def wrapped(q_r, kt_r, f1_r, f2_r, *wr):
        _fp_kernel(q_r, kt_r, f1_r, f2_r, *wr[:-1], out_ref=wr[-1])

    return pl.pallas_call(
        wrapped,
        grid=(B,),
        in_specs=[
            pl.BlockSpec((1, S, 3), lambda b: (b, 0, 0)),
            pl.BlockSpec((1, 3, K), lambda b: (b, 0, 0)),
            pl.BlockSpec((1, S, c1), lambda b: (b, 0, 0)),
            pl.BlockSpec((1, K, c2), lambda b: (b, 0, 0)),
        ] + wspecs,
        out_specs=pl.BlockSpec((1, S, cout), lambda b: (b, 0, 0)),
        out_shape=jax.ShapeDtypeStruct((B, S, cout), jnp.float32),
        interpret=_INTERPRET,
    )(unknown, knownt, feats1, feats2, *wargs)


def kernel(pc, params):
    xyz = pc[..., :3]
    inds1, c1 = _fps(xyz, 1024)
    grouped1 = _ball_gather_sc(
        xyz[:, :, 0], xyz[:, :, 1], xyz[:, :, 2],
        c1[:, :, 0].reshape(-1), c1[:, :, 1].reshape(-1),
        c1[:, :, 2].reshape(-1), 0.1, 64)
    f1 = _sa_from_grouped(grouped1, c1, 0.1, 64, params['sa1'], S_b=32)
    _, c2 = _fps(c1, 512)
    f2 = _sa(c1, f1, c2, 0.2, 32, params['sa2'], S_b=16, Pc=1024)
    _, c3 = _fps(c2, 256)
    f3 = _sa(c2, f2, c3, 0.4, 16, params['sa3'], S_b=32, Pc=512)
    latf = _fp(c2, c3, f2, f3, params['fp2'])
    return c2, latf, inds1[:, :512, 0]
